# Initial kernel scaffold; baseline (speedup 1.0000x reference)
import jax
import jax.numpy as jnp
from jax import lax
from jax.experimental import pallas as pl
from jax.experimental.pallas import tpu as pltpu
from jax.experimental.pallas import tpu_sc as plsc

NC = 2
NS = 16
L = 16
CH = 80


def _sc_scatter(x_hbm, src_hbm, dst_hbm, ew_hbm, zeros_hbm, hpart_hbm,
                src_v, dst_v, ew_v, rows_v, h_sp, sem):
    n, d = x_hbm.shape
    e = src_hbm.shape[0]
    nw = NC * NS
    epw = e // nw
    nch = epw // CH
    rpt = n // NS

    c = lax.axis_index("c")
    s = lax.axis_index("s")
    wid = s * NC + c

    row0 = s * rpt
    pltpu.sync_copy(zeros_hbm.at[pl.ds(row0, rpt)], h_sp.at[pl.ds(row0, rpt)])
    plsc.subcore_barrier()

    lanes = lax.iota(jnp.int32, L)

    def do_chunk(ci, carry):
        base = wid * epw + ci * CH
        pltpu.sync_copy(src_hbm.at[pl.ds(base, CH)], src_v)
        pltpu.sync_copy(dst_hbm.at[pl.ds(base, CH)], dst_v)
        pltpu.sync_copy(ew_hbm.at[pl.ds(base, CH)], ew_v)
        pltpu.async_copy(x_hbm.at[src_v], rows_v, sem).wait()

        def edge(i, cc):
            ii = jnp.full((L,), i, jnp.int32)
            wv = plsc.load_gather(ew_v, [ii])
            for j in range(d // L):
                col = lanes + (j * L)
                v = plsc.load_gather(rows_v, [ii, col])
                plsc.store_scatter(rows_v, [ii, col], v * wv)
            return cc

        lax.fori_loop(0, CH, edge, 0)
        pltpu.sync_copy(rows_v, h_sp.at[dst_v], add=True)
        return carry

    lax.fori_loop(0, nch, do_chunk, 0)
    plsc.subcore_barrier()

    pltpu.sync_copy(h_sp.at[pl.ds(row0, rpt)], hpart_hbm.at[c, pl.ds(row0, rpt)])


def _tc_linear(h_ref, w_ref, b_ref, o_ref):
    h = h_ref[0] + h_ref[1]
    o = lax.dot_general(h, w_ref[...], (((1,), (1,)), ((), ())),
                        preferred_element_type=jnp.float32)
    o_ref[...] = o + b_ref[...]


def kernel(x, edge_index, edge_weight, W, b):
    n, d = x.shape
    ei = edge_index.astype(jnp.int32)
    src = ei[0]
    dst = ei[1]
    zeros = jnp.zeros((n, d), jnp.float32)

    mesh = plsc.VectorSubcoreMesh(core_axis_name="c", subcore_axis_name="s")
    sc_call = pl.kernel(
        _sc_scatter,
        out_type=jax.ShapeDtypeStruct((NC, n, d), jnp.float32),
        mesh=mesh,
        scratch_types=[
            pltpu.VMEM((CH,), jnp.int32),
            pltpu.VMEM((CH,), jnp.int32),
            pltpu.VMEM((CH,), jnp.float32),
            pltpu.VMEM((CH, d), jnp.float32),
            pltpu.VMEM_SHARED((n, d), jnp.float32),
            pltpu.SemaphoreType.DMA,
        ],
    )
    hpart = sc_call(x, src, dst, edge_weight, zeros)

    bn = 1000
    out = pl.pallas_call(
        _tc_linear,
        grid=(n // bn,),
        in_specs=[
            pl.BlockSpec((NC, bn, d), lambda i: (0, i, 0)),
            pl.BlockSpec((d, d), lambda i: (0, 0)),
            pl.BlockSpec((1, d), lambda i: (0, 0)),
        ],
        out_specs=pl.BlockSpec((bn, d), lambda i: (i, 0)),
        out_shape=jax.ShapeDtypeStruct((n, d), jnp.float32),
    )(hpart, W, b.reshape(1, d))
    return out


# speedup vs baseline: 4.4791x; 4.4791x over previous
import jax
import jax.numpy as jnp
from jax import lax
from jax.experimental import pallas as pl
from jax.experimental.pallas import tpu as pltpu
from jax.experimental.pallas import tpu_sc as plsc

NC = 2
NS = 16
L = 16
CH = 80


def _sc_scatter(x_hbm, src_hbm, dst_hbm, ew_hbm, zeros_hbm, hpart_hbm,
                src_v, dst_v, ew_v, rows_v, h_sp, sem):
    npad, d = zeros_hbm.shape
    e = src_hbm.shape[0]
    nw = NC * NS
    epw = e // nw
    nch = epw // CH
    rpt = npad // NS

    c = lax.axis_index("c")
    s = lax.axis_index("s")
    wid = s * NC + c

    row0 = s * rpt
    pltpu.sync_copy(zeros_hbm.at[pl.ds(row0, rpt)], h_sp.at[pl.ds(row0, rpt)])
    plsc.subcore_barrier()

    lanes = lax.iota(jnp.int32, L)

    def do_chunk(ci, carry):
        base = wid * epw + ci * CH
        pltpu.sync_copy(src_hbm.at[pl.ds(base, CH)], src_v)
        pltpu.sync_copy(dst_hbm.at[pl.ds(base, CH)], dst_v)
        pltpu.sync_copy(ew_hbm.at[pl.ds(base, CH)], ew_v)
        pltpu.async_copy(x_hbm.at[src_v], rows_v, sem).wait()

        def edge_group(g, cc):
            ewv = ew_v[pl.ds(g * L, L)]
            for k in range(L):
                w = ewv[k]
                i = g * L + k
                for j in range(d // L):
                    v = rows_v[i, pl.ds(j * L, L)]
                    rows_v[i, pl.ds(j * L, L)] = v * w
            return cc

        lax.fori_loop(0, CH // L, edge_group, 0)
        pltpu.sync_copy(rows_v, h_sp.at[dst_v], add=True)
        return carry

    lax.fori_loop(0, nch, do_chunk, 0)
    plsc.subcore_barrier()

    pltpu.sync_copy(h_sp.at[pl.ds(row0, rpt)], hpart_hbm.at[c, pl.ds(row0, rpt)])


def _tc_linear(h_ref, w_ref, b_ref, o_ref):
    h = h_ref[0] + h_ref[1]
    o = lax.dot_general(h, w_ref[...], (((1,), (1,)), ((), ())),
                        preferred_element_type=jnp.float32)
    o_ref[...] = o + b_ref[...]


def kernel(x, edge_index, edge_weight, W, b):
    n, d = x.shape
    ei = edge_index.astype(jnp.int32)
    src = ei[0]
    dst = ei[1]
    npad = ((n + 8 * NS - 1) // (8 * NS)) * (8 * NS)
    zeros = jnp.zeros((npad, d), jnp.float32)

    mesh = plsc.VectorSubcoreMesh(core_axis_name="c", subcore_axis_name="s")
    sc_call = pl.kernel(
        _sc_scatter,
        out_type=jax.ShapeDtypeStruct((NC, npad, d), jnp.float32),
        mesh=mesh,
        scratch_types=[
            pltpu.VMEM((CH,), jnp.int32),
            pltpu.VMEM((CH,), jnp.int32),
            pltpu.VMEM((CH,), jnp.float32),
            pltpu.VMEM((CH, d), jnp.float32),
            pltpu.VMEM_SHARED((npad, d), jnp.float32),
            pltpu.SemaphoreType.DMA,
        ],
    )
    hpart = sc_call(x, src, dst, edge_weight, zeros)

    bn = 1000
    out = pl.pallas_call(
        _tc_linear,
        grid=(n // bn,),
        in_specs=[
            pl.BlockSpec((NC, bn, d), lambda i: (0, i, 0)),
            pl.BlockSpec((d, d), lambda i: (0, 0)),
            pl.BlockSpec((1, d), lambda i: (0, 0)),
        ],
        out_specs=pl.BlockSpec((bn, d), lambda i: (i, 0)),
        out_shape=jax.ShapeDtypeStruct((n, d), jnp.float32),
    )(hpart, W, b.reshape(1, d))
    return out


# speedup vs baseline: 8.8357x; 1.9726x over previous
import jax
import jax.numpy as jnp
from jax import lax
from jax.experimental import pallas as pl
from jax.experimental.pallas import tpu as pltpu
from jax.experimental.pallas import tpu_sc as plsc

NC = 2
NS = 16
L = 16
CH = 80
K = 4
KI = 8


def _sc_scatter(x_hbm, comb_hbm, ew_hbm, zeros_hbm, hpart_hbm,
                cbuf, ebuf, rows_v, h_sp, csem, esem, gsem, ssem):
    npad, d = zeros_hbm.shape
    nch = comb_hbm.shape[1]
    rpt = npad // NS

    c = lax.axis_index("c")
    s = lax.axis_index("s")
    wid = s * NC + c

    row0 = s * rpt
    pltpu.sync_copy(zeros_hbm.at[pl.ds(row0, rpt)], h_sp.at[pl.ds(row0, rpt)])
    plsc.subcore_barrier()

    def idx_start(ci, slot):
        pltpu.async_copy(comb_hbm.at[wid, ci], cbuf.at[slot], csem.at[slot])
        pltpu.async_copy(ew_hbm.at[wid, ci], ebuf.at[slot], esem.at[slot])

    def idx_wait(ci, slot):
        pltpu.make_async_copy(comb_hbm.at[wid, ci], cbuf.at[slot],
                              csem.at[slot]).wait()
        pltpu.make_async_copy(ew_hbm.at[wid, ci], ebuf.at[slot],
                              esem.at[slot]).wait()

    def gather_start(ci, b):
        pltpu.async_copy(x_hbm.at[cbuf.at[ci & (KI - 1), 0]], rows_v.at[b],
                         gsem.at[b])

    def gather_wait(ci, b):
        pltpu.make_async_copy(x_hbm.at[cbuf.at[ci & (KI - 1), 0]],
                              rows_v.at[b], gsem.at[b]).wait()

    def scatter_start(ci, b):
        pltpu.async_copy(rows_v.at[b], h_sp.at[cbuf.at[ci & (KI - 1), 1]],
                         ssem.at[b], add=True)

    def scatter_wait(ci, b):
        pltpu.make_async_copy(rows_v.at[b], h_sp.at[cbuf.at[ci & (KI - 1), 1]],
                              ssem.at[b]).wait()

    def scale_chunk(ci, b):
        slot = ci & (KI - 1)

        def tgroup(t, carry):
            ewvec = ebuf[slot, pl.ds(t * L, L)]
            for k in range(L):
                w = ewvec[k]
                i = t * L + k
                for j in range(d // L):
                    v = rows_v[b, i, pl.ds(j * L, L)]
                    rows_v[b, i, pl.ds(j * L, L)] = v * w
            return carry
        lax.fori_loop(0, CH // L, tgroup, 0, unroll=CH // L)

    for b in range(K):
        idx_start(b, b)
    for b in range(K):
        idx_wait(b, b)
        gather_start(b, b)

    def step(ci, b):
        gather_wait(ci, b)
        scale_chunk(ci, b)
        cprev = ci - 1
        bprev = (b - 1) % K

        @pl.when(cprev >= 0)
        def _():
            scatter_wait(cprev, bprev)

            @pl.when(cprev + K < nch)
            def _():
                idx_wait(cprev + K, (cprev + K) & (KI - 1))
                gather_start(cprev + K, bprev)

        scatter_start(ci, b)

        @pl.when(ci + K < nch)
        def _():
            idx_start(ci + K, (ci + K) & (KI - 1))

    def group(g, carry):
        for b in range(K):
            step(g * K + b, b)
        return carry

    ngroups = nch // K
    lax.fori_loop(0, ngroups, group, 0)
    for r in range(ngroups * K, nch):
        step(r, r % K)
    scatter_wait(nch - 1, (nch - 1) % K)
    plsc.subcore_barrier()

    pltpu.sync_copy(h_sp.at[pl.ds(row0, rpt)], hpart_hbm.at[c, pl.ds(row0, rpt)])


def _tc_linear(h_ref, w_ref, b_ref, o_ref):
    h = h_ref[0] + h_ref[1]
    o = lax.dot_general(h, w_ref[...], (((1,), (1,)), ((), ())),
                        preferred_element_type=jnp.float32)
    o_ref[...] = o + b_ref[...]


def kernel(x, edge_index, edge_weight, W, b):
    n, d = x.shape
    e = edge_weight.shape[0]
    nw = NC * NS
    epw = e // nw
    nch = epw // CH
    ei = edge_index.astype(jnp.int32)
    comb = jnp.stack(
        [ei[0].reshape(nw, nch, CH), ei[1].reshape(nw, nch, CH)], axis=2)
    ew3 = edge_weight.reshape(nw, nch, CH)
    npad = ((n + 8 * NS - 1) // (8 * NS)) * (8 * NS)
    zeros = jnp.zeros((npad, d), jnp.float32)

    mesh = plsc.VectorSubcoreMesh(core_axis_name="c", subcore_axis_name="s")
    sc_call = pl.kernel(
        _sc_scatter,
        out_type=jax.ShapeDtypeStruct((NC, npad, d), jnp.float32),
        mesh=mesh,
        scratch_types=[
            pltpu.VMEM((KI, 2, CH), jnp.int32),
            pltpu.VMEM((KI, CH), jnp.float32),
            pltpu.VMEM((K, CH, d), jnp.float32),
            pltpu.VMEM_SHARED((npad, d), jnp.float32),
            pltpu.SemaphoreType.DMA((KI,)),
            pltpu.SemaphoreType.DMA((KI,)),
            pltpu.SemaphoreType.DMA((K,)),
            pltpu.SemaphoreType.DMA((K,)),
        ],
    )
    hpart = sc_call(x, comb, ew3, zeros)

    bn = 1000
    out = pl.pallas_call(
        _tc_linear,
        grid=(n // bn,),
        in_specs=[
            pl.BlockSpec((NC, bn, d), lambda i: (0, i, 0)),
            pl.BlockSpec((d, d), lambda i: (0, 0)),
            pl.BlockSpec((1, d), lambda i: (0, 0)),
        ],
        out_specs=pl.BlockSpec((bn, d), lambda i: (i, 0)),
        out_shape=jax.ShapeDtypeStruct((n, d), jnp.float32),
    )(hpart, W, b.reshape(1, d))
    return out
